# Initial kernel scaffold; baseline (speedup 1.0000x reference)
import jax
import jax.numpy as jnp
from jax.experimental import pallas as pl


def kernel(x, lengths, conv_w, bn_gamma, bn_beta, bn_mean, bn_var, w_ih_l0, w_hh_l0, b_ih_l0, b_hh_l0, w_ih_l1, w_hh_l1, b_ih_l1, b_hh_l1, w_xy, b_xy, w_r, b_r):
    raise NotImplementedError("write your pallas kernel here")


# speedup vs baseline: 1.4090x; 1.4090x over previous
import functools

import jax
import jax.numpy as jnp
from jax import lax
from jax.experimental import pallas as pl
from jax.experimental.pallas import tpu as pltpu


def _fused_gru_kernel(len_ref,
                      y_ref,
                      wih0_ref,
                      b0_ref,
                      whh_ref,
                      wih1_ref,
                      bih1_ref,
                      bhn0_ref,
                      bhn1_ref,
                      whead_ref,
                      bhead_ref,
                      out_ref,
                      h1_sc, h2_sc,
                      gi0_sc,
                      h2a_sc,
                      *, Tc, H):
    tc = pl.program_id(1)

    @pl.when(tc == 0)
    def _():
        h1_sc[...] = jnp.zeros_like(h1_sc)
        h2_sc[...] = jnp.zeros_like(h2_sc)

    Bblk = y_ref.shape[1]
    lenc = len_ref[...]
    whh = whh_ref[...]
    wih1 = wih1_ref[...]
    bih1 = bih1_ref[...]
    bhn0 = bhn0_ref[...]
    bhn1 = bhn1_ref[...]
    t_base = tc * Tc

    yflat = y_ref[...].reshape(Tc * Bblk, H)
    gi0 = jnp.dot(yflat, wih0_ref[...], preferred_element_type=jnp.float32)
    gi0_sc[...] = (gi0 + b0_ref[...]).reshape(Tc, Bblk, 3 * H)

    def cell(gi, gh, bhn, h_prev):
        r = jax.nn.sigmoid(gi[:, :H] + gh[:, :H])
        z = jax.nn.sigmoid(gi[:, H:2 * H] + gh[:, H:2 * H])
        n = jnp.tanh(gi[:, 2 * H:] + r * (gh[:, 2 * H:] + bhn))
        return (1.0 - z) * n + z * h_prev

    def body(tt, carry):
        h1, h2 = carry
        valid = (t_base + tt) < lenc
        gi0t = gi0_sc[tt]
        hcat = jnp.concatenate([h1, h2], axis=-1)
        gh = jnp.dot(hcat, whh, preferred_element_type=jnp.float32)
        h1n = cell(gi0t, gh[:, :3 * H], bhn0, h1)
        gi1 = jnp.dot(h1n, wih1, preferred_element_type=jnp.float32) + bih1
        h2n = cell(gi1, gh[:, 3 * H:], bhn1, h2)
        h2a_sc[tt] = jnp.where(valid, h2n, 0.0)
        return (jnp.where(valid, h1n, h1), jnp.where(valid, h2n, h2))

    h1, h2 = lax.fori_loop(0, Tc, body, (h1_sc[...], h2_sc[...]), unroll=8)
    h1_sc[...] = h1
    h2_sc[...] = h2

    h2flat = h2a_sc[...].reshape(Tc * Bblk, H)
    lin = jnp.dot(h2flat, whead_ref[...], preferred_element_type=jnp.float32) + bhead_ref[...]
    col = lax.broadcasted_iota(jnp.int32, lin.shape, 1)
    outv = jnp.where((col >= 2) & (col < 4), jax.nn.softplus(lin), lin)
    out_ref[...] = outv.reshape(Tc, Bblk, 8)


def kernel(x, lengths, conv_w, bn_gamma, bn_beta, bn_mean, bn_var,
           w_ih_l0, w_hh_l0, b_ih_l0, b_hh_l0,
           w_ih_l1, w_hh_l1, b_ih_l1, b_hh_l1,
           w_xy, b_xy, w_r, b_r):
    B, T, Cin = x.shape
    H = conv_w.shape[0]
    eps = 1e-5

    xt = jnp.transpose(x.astype(jnp.float32), (1, 0, 2))
    xtp = jnp.pad(xt, ((1, 1), (0, 0), (0, 0)))
    y = jnp.einsum("tbc,oc->tbo", xtp[0:T], conv_w[:, :, 0])
    y = y + jnp.einsum("tbc,oc->tbo", xtp[1:T + 1], conv_w[:, :, 1])
    y = y + jnp.einsum("tbc,oc->tbo", xtp[2:T + 2], conv_w[:, :, 2])
    y = jnp.maximum(y, 0.0)

    scale = bn_gamma / jnp.sqrt(bn_var + eps)
    shift = bn_beta - bn_mean * scale
    wih0 = w_ih_l0.T
    wih0_eff = wih0 * scale[:, None]
    bias0 = b_ih_l0 + shift @ wih0
    bias0 = bias0.at[:2 * H].add(b_hh_l0[:2 * H])
    bias0 = bias0.reshape(1, 3 * H)
    bhn0 = b_hh_l0[2 * H:].reshape(1, H)

    wih1 = w_ih_l1.T
    bih1 = b_ih_l1.at[:2 * H].add(b_hh_l1[:2 * H]).reshape(1, 3 * H)
    bhn1 = b_hh_l1[2 * H:].reshape(1, H)

    whh = jnp.zeros((2 * H, 6 * H), jnp.float32)
    whh = whh.at[:H, :3 * H].set(w_hh_l0.T)
    whh = whh.at[H:, 3 * H:].set(w_hh_l1.T)

    whead = jnp.zeros((H, 8), jnp.float32)
    whead = whead.at[:, 0:2].set(w_xy.T)
    whead = whead.at[:, 2:4].set(w_r.T)
    bhead = jnp.zeros((8,), jnp.float32)
    bhead = bhead.at[0:2].set(b_xy)
    bhead = bhead.at[2:4].set(b_r)
    bhead = bhead.reshape(1, 8)

    Bblk = min(128, B)
    B_pad = ((B + Bblk - 1) // Bblk) * Bblk
    n_blocks = B_pad // Bblk
    Tc = min(64, T)
    T_pad = ((T + Tc - 1) // Tc) * Tc
    n_tc = T_pad // Tc

    y = jnp.pad(y, ((0, T_pad - T), (0, B_pad - B), (0, 0)))
    lengths_p = jnp.pad(lengths.astype(jnp.int32), (0, B_pad - B)).reshape(B_pad, 1)

    grid_spec = pltpu.PrefetchScalarGridSpec(
        num_scalar_prefetch=0,
        grid=(n_blocks, n_tc),
        in_specs=[
            pl.BlockSpec((Bblk, 1), lambda i, t: (i, 0)),
            pl.BlockSpec((Tc, Bblk, H), lambda i, t: (t, i, 0)),
            pl.BlockSpec((H, 3 * H), lambda i, t: (0, 0)),
            pl.BlockSpec((1, 3 * H), lambda i, t: (0, 0)),
            pl.BlockSpec((2 * H, 6 * H), lambda i, t: (0, 0)),
            pl.BlockSpec((H, 3 * H), lambda i, t: (0, 0)),
            pl.BlockSpec((1, 3 * H), lambda i, t: (0, 0)),
            pl.BlockSpec((1, H), lambda i, t: (0, 0)),
            pl.BlockSpec((1, H), lambda i, t: (0, 0)),
            pl.BlockSpec((H, 8), lambda i, t: (0, 0)),
            pl.BlockSpec((1, 8), lambda i, t: (0, 0)),
        ],
        out_specs=pl.BlockSpec((Tc, Bblk, 8), lambda i, t: (t, i, 0)),
        scratch_shapes=[pltpu.VMEM((Bblk, H), jnp.float32),
                        pltpu.VMEM((Bblk, H), jnp.float32),
                        pltpu.VMEM((Tc, Bblk, 3 * H), jnp.float32),
                        pltpu.VMEM((Tc, Bblk, H), jnp.float32)],
    )

    out8 = pl.pallas_call(
        functools.partial(_fused_gru_kernel, Tc=Tc, H=H),
        out_shape=jax.ShapeDtypeStruct((T_pad, B_pad, 8), jnp.float32),
        grid_spec=grid_spec,
        compiler_params=pltpu.CompilerParams(
            dimension_semantics=("parallel", "arbitrary"),
            vmem_limit_bytes=100 << 20,
        ),
    )(lengths_p, y, wih0_eff, bias0, whh, wih1, bih1, bhn0, bhn1, whead, bhead)

    return jnp.transpose(out8, (1, 0, 2))[:B, :T, :4]


# speedup vs baseline: 1.5943x; 1.1316x over previous
import functools

import jax
import jax.numpy as jnp
from jax import lax
from jax.experimental import pallas as pl
from jax.experimental.pallas import tpu as pltpu


def _fused_gru_kernel(len_ref,
                      y_ref,
                      wih0_ref,
                      b0_ref,
                      whh0_ref,
                      whh1_ref,
                      wih1_ref,
                      bih1_ref,
                      bhn0_ref,
                      bhn1_ref,
                      whead_ref,
                      bhead_ref,
                      out_ref,
                      h1_sc, h2_sc,
                      gi0_sc,
                      h2a_sc,
                      *, Tc, H):
    tc = pl.program_id(1)

    @pl.when(tc == 0)
    def _():
        h1_sc[...] = jnp.zeros_like(h1_sc)
        h2_sc[...] = jnp.zeros_like(h2_sc)

    Bblk = y_ref.shape[1]
    lenc = len_ref[...]
    whh0 = whh0_ref[...]
    whh1 = whh1_ref[...]
    wih1 = wih1_ref[...]
    bih1 = bih1_ref[...]
    bhn0 = bhn0_ref[...]
    bhn1 = bhn1_ref[...]
    t_base = tc * Tc
    bf = jnp.bfloat16

    yflat = y_ref[...].reshape(Tc * Bblk, H).astype(bf)
    gi0 = jnp.dot(yflat, wih0_ref[...], preferred_element_type=jnp.float32)
    gi0_sc[...] = (gi0 + b0_ref[...]).reshape(Tc, Bblk, 3 * H)

    def cell(gi, gh, bhn, h_prev):
        r = jax.nn.sigmoid(gi[:, :H] + gh[:, :H])
        z = jax.nn.sigmoid(gi[:, H:2 * H] + gh[:, H:2 * H])
        n = jnp.tanh(gi[:, 2 * H:] + r * (gh[:, 2 * H:] + bhn))
        return (1.0 - z) * n + z * h_prev

    def body(tt, carry):
        h1, h2 = carry
        valid = (t_base + tt) < lenc
        gi0t = gi0_sc[tt]
        gh0 = jnp.dot(h1.astype(bf), whh0, preferred_element_type=jnp.float32)
        gh1 = jnp.dot(h2.astype(bf), whh1, preferred_element_type=jnp.float32)
        h1n = cell(gi0t, gh0, bhn0, h1)
        gi1 = jnp.dot(h1n.astype(bf), wih1, preferred_element_type=jnp.float32) + bih1
        h2n = cell(gi1, gh1, bhn1, h2)
        h2a_sc[tt] = jnp.where(valid, h2n, 0.0)
        return (jnp.where(valid, h1n, h1), jnp.where(valid, h2n, h2))

    h1, h2 = lax.fori_loop(0, Tc, body, (h1_sc[...], h2_sc[...]), unroll=8)
    h1_sc[...] = h1
    h2_sc[...] = h2

    h2flat = h2a_sc[...].reshape(Tc * Bblk, H)
    lin = jnp.dot(h2flat, whead_ref[...], preferred_element_type=jnp.float32) + bhead_ref[...]
    col = lax.broadcasted_iota(jnp.int32, lin.shape, 1)
    outv = jnp.where((col >= 2) & (col < 4), jax.nn.softplus(lin), lin)
    out_ref[...] = outv.reshape(Tc, Bblk, 8)


def kernel(x, lengths, conv_w, bn_gamma, bn_beta, bn_mean, bn_var,
           w_ih_l0, w_hh_l0, b_ih_l0, b_hh_l0,
           w_ih_l1, w_hh_l1, b_ih_l1, b_hh_l1,
           w_xy, b_xy, w_r, b_r):
    B, T, Cin = x.shape
    H = conv_w.shape[0]
    eps = 1e-5

    xt = jnp.transpose(x.astype(jnp.float32), (1, 0, 2))
    xtp = jnp.pad(xt, ((1, 1), (0, 0), (0, 0)))
    y = jnp.einsum("tbc,oc->tbo", xtp[0:T], conv_w[:, :, 0])
    y = y + jnp.einsum("tbc,oc->tbo", xtp[1:T + 1], conv_w[:, :, 1])
    y = y + jnp.einsum("tbc,oc->tbo", xtp[2:T + 2], conv_w[:, :, 2])
    y = jnp.maximum(y, 0.0)

    scale = bn_gamma / jnp.sqrt(bn_var + eps)
    shift = bn_beta - bn_mean * scale
    wih0 = w_ih_l0.T
    wih0_eff = (wih0 * scale[:, None]).astype(jnp.bfloat16)
    bias0 = b_ih_l0 + shift @ wih0
    bias0 = bias0.at[:2 * H].add(b_hh_l0[:2 * H])
    bias0 = bias0.reshape(1, 3 * H)
    bhn0 = b_hh_l0[2 * H:].reshape(1, H)

    wih1 = w_ih_l1.T.astype(jnp.bfloat16)
    bih1 = b_ih_l1.at[:2 * H].add(b_hh_l1[:2 * H]).reshape(1, 3 * H)
    bhn1 = b_hh_l1[2 * H:].reshape(1, H)

    whh0 = w_hh_l0.T.astype(jnp.bfloat16)
    whh1 = w_hh_l1.T.astype(jnp.bfloat16)

    whead = jnp.zeros((H, 8), jnp.float32)
    whead = whead.at[:, 0:2].set(w_xy.T)
    whead = whead.at[:, 2:4].set(w_r.T)
    bhead = jnp.zeros((8,), jnp.float32)
    bhead = bhead.at[0:2].set(b_xy)
    bhead = bhead.at[2:4].set(b_r)
    bhead = bhead.reshape(1, 8)

    Bblk = min(128, B)
    B_pad = ((B + Bblk - 1) // Bblk) * Bblk
    n_blocks = B_pad // Bblk
    Tc = min(64, T)
    T_pad = ((T + Tc - 1) // Tc) * Tc
    n_tc = T_pad // Tc

    y = jnp.pad(y, ((0, T_pad - T), (0, B_pad - B), (0, 0)))
    lengths_p = jnp.pad(lengths.astype(jnp.int32), (0, B_pad - B)).reshape(B_pad, 1)

    grid_spec = pltpu.PrefetchScalarGridSpec(
        num_scalar_prefetch=0,
        grid=(n_blocks, n_tc),
        in_specs=[
            pl.BlockSpec((Bblk, 1), lambda i, t: (i, 0)),
            pl.BlockSpec((Tc, Bblk, H), lambda i, t: (t, i, 0)),
            pl.BlockSpec((H, 3 * H), lambda i, t: (0, 0)),
            pl.BlockSpec((1, 3 * H), lambda i, t: (0, 0)),
            pl.BlockSpec((H, 3 * H), lambda i, t: (0, 0)),
            pl.BlockSpec((H, 3 * H), lambda i, t: (0, 0)),
            pl.BlockSpec((H, 3 * H), lambda i, t: (0, 0)),
            pl.BlockSpec((1, 3 * H), lambda i, t: (0, 0)),
            pl.BlockSpec((1, H), lambda i, t: (0, 0)),
            pl.BlockSpec((1, H), lambda i, t: (0, 0)),
            pl.BlockSpec((H, 8), lambda i, t: (0, 0)),
            pl.BlockSpec((1, 8), lambda i, t: (0, 0)),
        ],
        out_specs=pl.BlockSpec((Tc, Bblk, 8), lambda i, t: (t, i, 0)),
        scratch_shapes=[pltpu.VMEM((Bblk, H), jnp.float32),
                        pltpu.VMEM((Bblk, H), jnp.float32),
                        pltpu.VMEM((Tc, Bblk, 3 * H), jnp.float32),
                        pltpu.VMEM((Tc, Bblk, H), jnp.float32)],
    )

    out8 = pl.pallas_call(
        functools.partial(_fused_gru_kernel, Tc=Tc, H=H),
        out_shape=jax.ShapeDtypeStruct((T_pad, B_pad, 8), jnp.float32),
        grid_spec=grid_spec,
        compiler_params=pltpu.CompilerParams(
            dimension_semantics=("parallel", "arbitrary"),
            vmem_limit_bytes=100 << 20,
        ),
    )(lengths_p, y, wih0_eff, bias0, whh0, whh1, wih1, bih1, bhn0, bhn1, whead, bhead)

    return jnp.transpose(out8, (1, 0, 2))[:B, :T, :4]


# speedup vs baseline: 1.6600x; 1.0412x over previous
import functools

import jax
import jax.numpy as jnp
from jax import lax
from jax.experimental import pallas as pl
from jax.experimental.pallas import tpu as pltpu


def _fused_gru_kernel(len_ref,
                      y_ref,
                      wih0_ref,
                      b0_ref,
                      whh0_ref,
                      whh1_ref,
                      wih1_ref,
                      bih1_ref,
                      bhn0_ref,
                      bhn1_ref,
                      whead_ref,
                      bhead_ref,
                      out_ref,
                      h1_sc, h2_sc,
                      gi0_sc,
                      h2a_sc,
                      *, Tc, H):
    tc = pl.program_id(1)

    @pl.when(tc == 0)
    def _():
        h1_sc[...] = jnp.zeros_like(h1_sc)
        h2_sc[...] = jnp.zeros_like(h2_sc)

    Bblk = y_ref.shape[1]
    lenc = len_ref[...]
    whh0 = whh0_ref[...]
    whh1 = whh1_ref[...]
    wih1 = wih1_ref[...]
    bih1 = bih1_ref[...]
    bhn0 = bhn0_ref[...]
    bhn1 = bhn1_ref[...]
    t_base = tc * Tc
    bf = jnp.bfloat16

    yflat = y_ref[...].reshape(Tc * Bblk, H).astype(bf)
    gi0 = jnp.dot(yflat, wih0_ref[...], preferred_element_type=jnp.float32)
    gi0_sc[...] = (gi0 + b0_ref[...]).reshape(Tc, Bblk, 3 * H)

    def cell(gi, gh, bhn, h_prev):
        r = jax.nn.sigmoid(gi[:, :H] + gh[:, :H])
        z = jax.nn.sigmoid(gi[:, H:2 * H] + gh[:, H:2 * H])
        n = jnp.tanh(gi[:, 2 * H:] + r * (gh[:, 2 * H:] + bhn))
        return (1.0 - z) * n + z * h_prev

    def body(tt, carry):
        h1, h2 = carry
        valid = (t_base + tt) < lenc
        gi0t = gi0_sc[tt]
        gh0 = jnp.dot(h1.astype(bf), whh0, preferred_element_type=jnp.float32)
        gh1 = jnp.dot(h2.astype(bf), whh1, preferred_element_type=jnp.float32)
        h1n = cell(gi0t, gh0, bhn0, h1)
        gi1 = jnp.dot(h1n.astype(bf), wih1, preferred_element_type=jnp.float32) + bih1
        h2n = cell(gi1, gh1, bhn1, h2)
        h2a_sc[tt] = jnp.where(valid, h2n, 0.0)
        return (jnp.where(valid, h1n, h1), jnp.where(valid, h2n, h2))

    h1, h2 = lax.fori_loop(0, Tc, body, (h1_sc[...], h2_sc[...]), unroll=8)
    h1_sc[...] = h1
    h2_sc[...] = h2

    h2flat = h2a_sc[...].reshape(Tc * Bblk, H)
    lin = jnp.dot(h2flat, whead_ref[...], preferred_element_type=jnp.float32) + bhead_ref[...]
    col = lax.broadcasted_iota(jnp.int32, lin.shape, 1)
    outv = jnp.where((col >= 2) & (col < 4), jax.nn.softplus(lin), lin)
    out_ref[...] = outv.reshape(Tc, Bblk, 8)


def kernel(x, lengths, conv_w, bn_gamma, bn_beta, bn_mean, bn_var,
           w_ih_l0, w_hh_l0, b_ih_l0, b_hh_l0,
           w_ih_l1, w_hh_l1, b_ih_l1, b_hh_l1,
           w_xy, b_xy, w_r, b_r):
    B, T, Cin = x.shape
    H = conv_w.shape[0]
    eps = 1e-5

    xt = jnp.transpose(x.astype(jnp.bfloat16), (1, 0, 2))
    xtp = jnp.pad(xt, ((1, 1), (0, 0), (0, 0)))
    cw = conv_w.astype(jnp.bfloat16)
    y = jnp.einsum("tbc,oc->tbo", xtp[0:T], cw[:, :, 0],
                   preferred_element_type=jnp.float32)
    y = y + jnp.einsum("tbc,oc->tbo", xtp[1:T + 1], cw[:, :, 1],
                       preferred_element_type=jnp.float32)
    y = y + jnp.einsum("tbc,oc->tbo", xtp[2:T + 2], cw[:, :, 2],
                       preferred_element_type=jnp.float32)
    y = jnp.maximum(y, 0.0).astype(jnp.bfloat16)

    scale = bn_gamma / jnp.sqrt(bn_var + eps)
    shift = bn_beta - bn_mean * scale
    wih0 = w_ih_l0.T
    wih0_eff = (wih0 * scale[:, None]).astype(jnp.bfloat16)
    bias0 = b_ih_l0 + shift @ wih0
    bias0 = bias0.at[:2 * H].add(b_hh_l0[:2 * H])
    bias0 = bias0.reshape(1, 3 * H)
    bhn0 = b_hh_l0[2 * H:].reshape(1, H)

    wih1 = w_ih_l1.T.astype(jnp.bfloat16)
    bih1 = b_ih_l1.at[:2 * H].add(b_hh_l1[:2 * H]).reshape(1, 3 * H)
    bhn1 = b_hh_l1[2 * H:].reshape(1, H)

    whh0 = w_hh_l0.T.astype(jnp.bfloat16)
    whh1 = w_hh_l1.T.astype(jnp.bfloat16)

    whead = jnp.zeros((H, 8), jnp.float32)
    whead = whead.at[:, 0:2].set(w_xy.T)
    whead = whead.at[:, 2:4].set(w_r.T)
    bhead = jnp.zeros((8,), jnp.float32)
    bhead = bhead.at[0:2].set(b_xy)
    bhead = bhead.at[2:4].set(b_r)
    bhead = bhead.reshape(1, 8)

    Bblk = min(128, B)
    B_pad = ((B + Bblk - 1) // Bblk) * Bblk
    n_blocks = B_pad // Bblk
    Tc = min(64, T)
    T_pad = ((T + Tc - 1) // Tc) * Tc
    n_tc = T_pad // Tc

    y = jnp.pad(y, ((0, T_pad - T), (0, B_pad - B), (0, 0)))
    lengths_p = jnp.pad(lengths.astype(jnp.int32), (0, B_pad - B)).reshape(B_pad, 1)

    grid_spec = pltpu.PrefetchScalarGridSpec(
        num_scalar_prefetch=0,
        grid=(n_blocks, n_tc),
        in_specs=[
            pl.BlockSpec((Bblk, 1), lambda i, t: (i, 0)),
            pl.BlockSpec((Tc, Bblk, H), lambda i, t: (t, i, 0)),
            pl.BlockSpec((H, 3 * H), lambda i, t: (0, 0)),
            pl.BlockSpec((1, 3 * H), lambda i, t: (0, 0)),
            pl.BlockSpec((H, 3 * H), lambda i, t: (0, 0)),
            pl.BlockSpec((H, 3 * H), lambda i, t: (0, 0)),
            pl.BlockSpec((H, 3 * H), lambda i, t: (0, 0)),
            pl.BlockSpec((1, 3 * H), lambda i, t: (0, 0)),
            pl.BlockSpec((1, H), lambda i, t: (0, 0)),
            pl.BlockSpec((1, H), lambda i, t: (0, 0)),
            pl.BlockSpec((H, 8), lambda i, t: (0, 0)),
            pl.BlockSpec((1, 8), lambda i, t: (0, 0)),
        ],
        out_specs=pl.BlockSpec((Tc, Bblk, 8), lambda i, t: (t, i, 0)),
        scratch_shapes=[pltpu.VMEM((Bblk, H), jnp.float32),
                        pltpu.VMEM((Bblk, H), jnp.float32),
                        pltpu.VMEM((Tc, Bblk, 3 * H), jnp.float32),
                        pltpu.VMEM((Tc, Bblk, H), jnp.float32)],
    )

    out8 = pl.pallas_call(
        functools.partial(_fused_gru_kernel, Tc=Tc, H=H),
        out_shape=jax.ShapeDtypeStruct((T_pad, B_pad, 8), jnp.float32),
        grid_spec=grid_spec,
        compiler_params=pltpu.CompilerParams(
            dimension_semantics=("parallel", "arbitrary"),
            vmem_limit_bytes=100 << 20,
        ),
    )(lengths_p, y, wih0_eff, bias0, whh0, whh1, wih1, bih1, bhn0, bhn1, whead, bhead)

    return jnp.transpose(out8, (1, 0, 2))[:B, :T, :4]


# speedup vs baseline: 1.9041x; 1.1471x over previous
import functools

import jax
import jax.numpy as jnp
from jax import lax
from jax.experimental import pallas as pl
from jax.experimental.pallas import tpu as pltpu


def _fused_kernel(len_ref,
                  x_ref,
                  xh_ref,
                  wc0_ref, wc1_ref, wc2_ref,
                  wih0_ref,
                  b0_ref,
                  whh0_ref,
                  whh1_ref,
                  wih1_ref,
                  bih1_ref,
                  bhn0_ref,
                  bhn1_ref,
                  whead_ref,
                  bhead_ref,
                  out_ref,
                  h1_sc, h2_sc,
                  gi0_sc,
                  h2a_sc,
                  *, Tc, H):
    tc = pl.program_id(1)

    @pl.when(tc == 0)
    def _():
        h1_sc[...] = jnp.zeros_like(h1_sc)
        h2_sc[...] = jnp.zeros_like(h2_sc)

    Bblk = x_ref.shape[1]
    C = x_ref.shape[2]
    lenc = len_ref[...]
    whh0 = whh0_ref[...]
    whh1 = whh1_ref[...]
    wih1 = wih1_ref[...]
    bih1 = bih1_ref[...]
    bhn0 = bhn0_ref[...]
    bhn1 = bhn1_ref[...]
    t_base = tc * Tc
    bf = jnp.bfloat16

    xb = x_ref[...]
    xm1 = jnp.concatenate([xh_ref[0, 0][None], xb[:-1]], axis=0)
    xp1 = jnp.concatenate([xb[1:], xh_ref[0, 1][None]], axis=0)
    acc = jnp.dot(xm1.reshape(Tc * Bblk, C), wc0_ref[...],
                  preferred_element_type=jnp.float32)
    acc = acc + jnp.dot(xb.reshape(Tc * Bblk, C), wc1_ref[...],
                        preferred_element_type=jnp.float32)
    acc = acc + jnp.dot(xp1.reshape(Tc * Bblk, C), wc2_ref[...],
                        preferred_element_type=jnp.float32)
    y = jnp.maximum(acc, 0.0).astype(bf)
    gi0 = jnp.dot(y, wih0_ref[...], preferred_element_type=jnp.float32)
    gi0_sc[...] = (gi0 + b0_ref[...]).reshape(Tc, Bblk, 3 * H)

    def cell(gi, gh, bhn, h_prev):
        r = jax.nn.sigmoid(gi[:, :H] + gh[:, :H])
        z = jax.nn.sigmoid(gi[:, H:2 * H] + gh[:, H:2 * H])
        n = jnp.tanh(gi[:, 2 * H:] + r * (gh[:, 2 * H:] + bhn))
        return (1.0 - z) * n + z * h_prev

    def body(tt, carry):
        h1, h2 = carry
        valid = (t_base + tt) < lenc
        gi0t = gi0_sc[tt]
        gh0 = jnp.dot(h1.astype(bf), whh0, preferred_element_type=jnp.float32)
        gh1 = jnp.dot(h2.astype(bf), whh1, preferred_element_type=jnp.float32)
        h1n = cell(gi0t, gh0, bhn0, h1)
        gi1 = jnp.dot(h1n.astype(bf), wih1, preferred_element_type=jnp.float32) + bih1
        h2n = cell(gi1, gh1, bhn1, h2)
        h2a_sc[tt] = jnp.where(valid, h2n, 0.0)
        return (jnp.where(valid, h1n, h1), jnp.where(valid, h2n, h2))

    h1, h2 = lax.fori_loop(0, Tc, body, (h1_sc[...], h2_sc[...]), unroll=8)
    h1_sc[...] = h1
    h2_sc[...] = h2

    h2flat = h2a_sc[...].reshape(Tc * Bblk, H)
    lin = jnp.dot(h2flat, whead_ref[...], preferred_element_type=jnp.float32) + bhead_ref[...]
    col = lax.broadcasted_iota(jnp.int32, lin.shape, 1)
    outv = jnp.where((col >= 2) & (col < 4), jax.nn.softplus(lin), lin)
    out_ref[...] = outv.reshape(Tc, Bblk, 8)


def kernel(x, lengths, conv_w, bn_gamma, bn_beta, bn_mean, bn_var,
           w_ih_l0, w_hh_l0, b_ih_l0, b_hh_l0,
           w_ih_l1, w_hh_l1, b_ih_l1, b_hh_l1,
           w_xy, b_xy, w_r, b_r):
    B, T, Cin = x.shape
    H = conv_w.shape[0]
    eps = 1e-5

    Bblk = min(128, B)
    B_pad = ((B + Bblk - 1) // Bblk) * Bblk
    n_blocks = B_pad // Bblk
    Tc = min(64, T)
    T_pad = ((T + Tc - 1) // Tc) * Tc
    n_tc = T_pad // Tc

    xt = jnp.transpose(x.astype(jnp.bfloat16), (1, 0, 2))
    xt = jnp.pad(xt, ((0, T_pad - T), (0, B_pad - B), (0, 0)))
    zrow = jnp.zeros((1, B_pad, Cin), jnp.bfloat16)
    xprev = jnp.concatenate([zrow, xt[Tc - 1:T_pad - 1:Tc]], axis=0)
    xnext = jnp.concatenate([xt[Tc::Tc], zrow], axis=0)
    xhalo = jnp.stack([xprev, xnext], axis=1)

    wc0 = conv_w[:, :, 0].T.astype(jnp.bfloat16)
    wc1 = conv_w[:, :, 1].T.astype(jnp.bfloat16)
    wc2 = conv_w[:, :, 2].T.astype(jnp.bfloat16)

    scale = bn_gamma / jnp.sqrt(bn_var + eps)
    shift = bn_beta - bn_mean * scale
    wih0 = w_ih_l0.T
    wih0_eff = (wih0 * scale[:, None]).astype(jnp.bfloat16)
    bias0 = b_ih_l0 + shift @ wih0
    bias0 = bias0.at[:2 * H].add(b_hh_l0[:2 * H])
    bias0 = bias0.reshape(1, 3 * H)
    bhn0 = b_hh_l0[2 * H:].reshape(1, H)

    wih1 = w_ih_l1.T.astype(jnp.bfloat16)
    bih1 = b_ih_l1.at[:2 * H].add(b_hh_l1[:2 * H]).reshape(1, 3 * H)
    bhn1 = b_hh_l1[2 * H:].reshape(1, H)

    whh0 = w_hh_l0.T.astype(jnp.bfloat16)
    whh1 = w_hh_l1.T.astype(jnp.bfloat16)

    whead = jnp.zeros((H, 8), jnp.float32)
    whead = whead.at[:, 0:2].set(w_xy.T)
    whead = whead.at[:, 2:4].set(w_r.T)
    bhead = jnp.zeros((8,), jnp.float32)
    bhead = bhead.at[0:2].set(b_xy)
    bhead = bhead.at[2:4].set(b_r)
    bhead = bhead.reshape(1, 8)

    lengths_p = jnp.pad(lengths.astype(jnp.int32), (0, B_pad - B)).reshape(B_pad, 1)

    grid_spec = pltpu.PrefetchScalarGridSpec(
        num_scalar_prefetch=0,
        grid=(n_blocks, n_tc),
        in_specs=[
            pl.BlockSpec((Bblk, 1), lambda i, t: (i, 0)),
            pl.BlockSpec((Tc, Bblk, Cin), lambda i, t: (t, i, 0)),
            pl.BlockSpec((1, 2, Bblk, Cin), lambda i, t: (t, 0, i, 0)),
            pl.BlockSpec((Cin, H), lambda i, t: (0, 0)),
            pl.BlockSpec((Cin, H), lambda i, t: (0, 0)),
            pl.BlockSpec((Cin, H), lambda i, t: (0, 0)),
            pl.BlockSpec((H, 3 * H), lambda i, t: (0, 0)),
            pl.BlockSpec((1, 3 * H), lambda i, t: (0, 0)),
            pl.BlockSpec((H, 3 * H), lambda i, t: (0, 0)),
            pl.BlockSpec((H, 3 * H), lambda i, t: (0, 0)),
            pl.BlockSpec((H, 3 * H), lambda i, t: (0, 0)),
            pl.BlockSpec((1, 3 * H), lambda i, t: (0, 0)),
            pl.BlockSpec((1, H), lambda i, t: (0, 0)),
            pl.BlockSpec((1, H), lambda i, t: (0, 0)),
            pl.BlockSpec((H, 8), lambda i, t: (0, 0)),
            pl.BlockSpec((1, 8), lambda i, t: (0, 0)),
        ],
        out_specs=pl.BlockSpec((Tc, Bblk, 8), lambda i, t: (t, i, 0)),
        scratch_shapes=[pltpu.VMEM((Bblk, H), jnp.float32),
                        pltpu.VMEM((Bblk, H), jnp.float32),
                        pltpu.VMEM((Tc, Bblk, 3 * H), jnp.float32),
                        pltpu.VMEM((Tc, Bblk, H), jnp.float32)],
    )

    out8 = pl.pallas_call(
        functools.partial(_fused_kernel, Tc=Tc, H=H),
        out_shape=jax.ShapeDtypeStruct((T_pad, B_pad, 8), jnp.float32),
        grid_spec=grid_spec,
        compiler_params=pltpu.CompilerParams(
            dimension_semantics=("parallel", "arbitrary"),
            vmem_limit_bytes=100 << 20,
        ),
    )(lengths_p, xt, xhalo, wc0, wc1, wc2, wih0_eff, bias0,
      whh0, whh1, wih1, bih1, bhn0, bhn1, whead, bhead)

    return jnp.transpose(out8, (1, 0, 2))[:B, :T, :4]


# speedup vs baseline: 2.1105x; 1.1084x over previous
import functools

import jax
import jax.numpy as jnp
from jax import lax
from jax.experimental import pallas as pl
from jax.experimental.pallas import tpu as pltpu


def _fused_kernel(len_ref,
                  x_ref,
                  xh_ref,
                  wc0_ref, wc1_ref, wc2_ref,
                  wih0_ref,
                  b0_ref,
                  whh0_ref,
                  whh1_ref,
                  wih1_ref,
                  bih1_ref,
                  bhn0_ref,
                  bhn1_ref,
                  whead_ref,
                  bhead_ref,
                  out_ref,
                  h1_sc, h2_sc,
                  gi0_sc,
                  h2a_sc,
                  *, Tc, H):
    tc = pl.program_id(1)

    @pl.when(tc == 0)
    def _():
        h1_sc[...] = jnp.zeros_like(h1_sc)
        h2_sc[...] = jnp.zeros_like(h2_sc)

    Bblk = x_ref.shape[1]
    C = x_ref.shape[2]
    lenc = len_ref[...]
    whh0 = whh0_ref[...]
    whh1 = whh1_ref[...]
    wih1 = wih1_ref[...]
    bih1 = bih1_ref[...]
    bhn0 = bhn0_ref[...]
    bhn1 = bhn1_ref[...]
    t_base = tc * Tc
    bf = jnp.bfloat16

    xb = x_ref[...]
    xm1 = jnp.concatenate([xh_ref[0, 0][None], xb[:-1]], axis=0)
    xp1 = jnp.concatenate([xb[1:], xh_ref[0, 1][None]], axis=0)
    acc = jnp.dot(xm1.reshape(Tc * Bblk, C), wc0_ref[...],
                  preferred_element_type=jnp.float32)
    acc = acc + jnp.dot(xb.reshape(Tc * Bblk, C), wc1_ref[...],
                        preferred_element_type=jnp.float32)
    acc = acc + jnp.dot(xp1.reshape(Tc * Bblk, C), wc2_ref[...],
                        preferred_element_type=jnp.float32)
    y = jnp.maximum(acc, 0.0).astype(bf)
    gi0 = jnp.dot(y, wih0_ref[...], preferred_element_type=jnp.float32)
    gi0_sc[...] = (gi0 + b0_ref[...]).reshape(Tc, Bblk, 3 * H)

    def cell(gi, gh, bhn, h_prev):
        r = jax.nn.sigmoid(gi[:, :H] + gh[:, :H])
        z = jax.nn.sigmoid(gi[:, H:2 * H] + gh[:, H:2 * H])
        n = jnp.tanh(gi[:, 2 * H:] + r * (gh[:, 2 * H:] + bhn))
        return (1.0 - z) * n + z * h_prev

    def l0_step(tt, h1):
        valid = (t_base + tt) < lenc
        h1n = cell(gi0_sc[tt],
                   jnp.dot(h1.astype(bf), whh0, preferred_element_type=jnp.float32),
                   bhn0, h1)
        return h1n, jnp.where(valid, h1n, h1)

    def l1_step(tt, h1n, h2):
        valid = (t_base + tt) < lenc
        gi1 = jnp.dot(h1n.astype(bf), wih1, preferred_element_type=jnp.float32) + bih1
        gh1 = jnp.dot(h2.astype(bf), whh1, preferred_element_type=jnp.float32)
        h2n = cell(gi1, gh1, bhn1, h2)
        h2a_sc[tt] = jnp.where(valid, h2n, 0.0)
        return jnp.where(valid, h2n, h2)

    h1n0, h1_0 = l0_step(0, h1_sc[...])

    def body(ss, carry):
        h1, h2, h1n_prev = carry
        h1n, h1m = l0_step(ss, h1)
        h2m = l1_step(ss - 1, h1n_prev, h2)
        return (h1m, h2m, h1n)

    h1, h2, h1n_last = lax.fori_loop(1, Tc, body, (h1_0, h2_sc[...], h1n0),
                                     unroll=8)
    h2 = l1_step(Tc - 1, h1n_last, h2)
    h1_sc[...] = h1
    h2_sc[...] = h2

    h2flat = h2a_sc[...].reshape(Tc * Bblk, H)
    lin = jnp.dot(h2flat, whead_ref[...], preferred_element_type=jnp.float32) + bhead_ref[...]
    col = lax.broadcasted_iota(jnp.int32, lin.shape, 1)
    outv = jnp.where((col >= 2) & (col < 4), jax.nn.softplus(lin), lin)
    out_ref[...] = outv.reshape(Tc, Bblk, 8)


def kernel(x, lengths, conv_w, bn_gamma, bn_beta, bn_mean, bn_var,
           w_ih_l0, w_hh_l0, b_ih_l0, b_hh_l0,
           w_ih_l1, w_hh_l1, b_ih_l1, b_hh_l1,
           w_xy, b_xy, w_r, b_r):
    B, T, Cin = x.shape
    H = conv_w.shape[0]
    eps = 1e-5

    Bblk = min(128, B)
    B_pad = ((B + Bblk - 1) // Bblk) * Bblk
    n_blocks = B_pad // Bblk
    Tc = min(64, T)
    T_pad = ((T + Tc - 1) // Tc) * Tc
    n_tc = T_pad // Tc

    xt = jnp.transpose(x.astype(jnp.bfloat16), (1, 0, 2))
    xt = jnp.pad(xt, ((0, T_pad - T), (0, B_pad - B), (0, 0)))
    zrow = jnp.zeros((1, B_pad, Cin), jnp.bfloat16)
    xprev = jnp.concatenate([zrow, xt[Tc - 1:T_pad - 1:Tc]], axis=0)
    xnext = jnp.concatenate([xt[Tc::Tc], zrow], axis=0)
    xhalo = jnp.stack([xprev, xnext], axis=1)

    wc0 = conv_w[:, :, 0].T.astype(jnp.bfloat16)
    wc1 = conv_w[:, :, 1].T.astype(jnp.bfloat16)
    wc2 = conv_w[:, :, 2].T.astype(jnp.bfloat16)

    scale = bn_gamma / jnp.sqrt(bn_var + eps)
    shift = bn_beta - bn_mean * scale
    wih0 = w_ih_l0.T
    wih0_eff = (wih0 * scale[:, None]).astype(jnp.bfloat16)
    bias0 = b_ih_l0 + shift @ wih0
    bias0 = bias0.at[:2 * H].add(b_hh_l0[:2 * H])
    bias0 = bias0.reshape(1, 3 * H)
    bhn0 = b_hh_l0[2 * H:].reshape(1, H)

    wih1 = w_ih_l1.T.astype(jnp.bfloat16)
    bih1 = b_ih_l1.at[:2 * H].add(b_hh_l1[:2 * H]).reshape(1, 3 * H)
    bhn1 = b_hh_l1[2 * H:].reshape(1, H)

    whh0 = w_hh_l0.T.astype(jnp.bfloat16)
    whh1 = w_hh_l1.T.astype(jnp.bfloat16)

    whead = jnp.zeros((H, 8), jnp.float32)
    whead = whead.at[:, 0:2].set(w_xy.T)
    whead = whead.at[:, 2:4].set(w_r.T)
    bhead = jnp.zeros((8,), jnp.float32)
    bhead = bhead.at[0:2].set(b_xy)
    bhead = bhead.at[2:4].set(b_r)
    bhead = bhead.reshape(1, 8)

    lengths_p = jnp.pad(lengths.astype(jnp.int32), (0, B_pad - B)).reshape(B_pad, 1)

    grid_spec = pltpu.PrefetchScalarGridSpec(
        num_scalar_prefetch=0,
        grid=(n_blocks, n_tc),
        in_specs=[
            pl.BlockSpec((Bblk, 1), lambda i, t: (i, 0)),
            pl.BlockSpec((Tc, Bblk, Cin), lambda i, t: (t, i, 0)),
            pl.BlockSpec((1, 2, Bblk, Cin), lambda i, t: (t, 0, i, 0)),
            pl.BlockSpec((Cin, H), lambda i, t: (0, 0)),
            pl.BlockSpec((Cin, H), lambda i, t: (0, 0)),
            pl.BlockSpec((Cin, H), lambda i, t: (0, 0)),
            pl.BlockSpec((H, 3 * H), lambda i, t: (0, 0)),
            pl.BlockSpec((1, 3 * H), lambda i, t: (0, 0)),
            pl.BlockSpec((H, 3 * H), lambda i, t: (0, 0)),
            pl.BlockSpec((H, 3 * H), lambda i, t: (0, 0)),
            pl.BlockSpec((H, 3 * H), lambda i, t: (0, 0)),
            pl.BlockSpec((1, 3 * H), lambda i, t: (0, 0)),
            pl.BlockSpec((1, H), lambda i, t: (0, 0)),
            pl.BlockSpec((1, H), lambda i, t: (0, 0)),
            pl.BlockSpec((H, 8), lambda i, t: (0, 0)),
            pl.BlockSpec((1, 8), lambda i, t: (0, 0)),
        ],
        out_specs=pl.BlockSpec((Tc, Bblk, 8), lambda i, t: (t, i, 0)),
        scratch_shapes=[pltpu.VMEM((Bblk, H), jnp.float32),
                        pltpu.VMEM((Bblk, H), jnp.float32),
                        pltpu.VMEM((Tc, Bblk, 3 * H), jnp.float32),
                        pltpu.VMEM((Tc, Bblk, H), jnp.float32)],
    )

    out8 = pl.pallas_call(
        functools.partial(_fused_kernel, Tc=Tc, H=H),
        out_shape=jax.ShapeDtypeStruct((T_pad, B_pad, 8), jnp.float32),
        grid_spec=grid_spec,
        compiler_params=pltpu.CompilerParams(
            dimension_semantics=("parallel", "arbitrary"),
            vmem_limit_bytes=100 << 20,
        ),
    )(lengths_p, xt, xhalo, wc0, wc1, wc2, wih0_eff, bias0,
      whh0, whh1, wih1, bih1, bhn0, bhn1, whead, bhead)

    return jnp.transpose(out8, (1, 0, 2))[:B, :T, :4]
